# baseline (speedup 1.0000x reference)
import functools

import jax
import jax.numpy as jnp
from jax import lax
from jax.experimental import pallas as pl
from jax.experimental.pallas import tpu as pltpu
from jax.experimental.pallas import tpu_sc as plsc

EPS = 1e-12

_NC = 2
_NS = 16
_NW = _NC * _NS
_CHUNK = 16
_NBUF = 4
_BT = 2048


def _sc_gather_pack(ids_flat, table):
    T = ids_flat.shape[0]
    H = table.shape[1]
    Hp = H // 2
    per_w = T // _NW
    n_chunks = per_w // _CHUNK
    n_outer = n_chunks // _NBUF
    n_grp = Hp // 16
    mesh = plsc.VectorSubcoreMesh(core_axis_name="c", subcore_axis_name="s")

    @functools.partial(
        pl.kernel,
        out_type=jax.ShapeDtypeStruct((T, Hp), jnp.uint32),
        mesh=mesh,
        scratch_types=[
            pltpu.VMEM((per_w,), jnp.int32),
            pltpu.VMEM((_NBUF, _CHUNK, H), jnp.uint32),
            pltpu.VMEM((_NBUF, _CHUNK, Hp), jnp.uint32),
        ]
        + [pltpu.SemaphoreType.DMA] * (2 * _NBUF),
    )
    def k(ids_hbm, tab_hbm, out_hbm, idx_v, rows_v, pack_v, *sems):
        wid = lax.axis_index("s") * _NC + lax.axis_index("c")
        base = wid * per_w
        gsems = sems[:_NBUF]
        ssems = sems[_NBUF:]
        pltpu.sync_copy(ids_hbm.at[pl.ds(base, per_w)], idx_v)
        for b in range(_NBUF):
            pltpu.async_copy(
                tab_hbm.at[idx_v.at[pl.ds(b * _CHUNK, _CHUNK)]], rows_v.at[b], gsems[b]
            )

        def pack_chunk(rows_ref, pack_ref):
            def row_body(r, carry):
                for j in range(n_grp):
                    yu = rows_ref[r, pl.ds(16 * j, 16)] + jnp.uint32(0x8000)
                    yv = rows_ref[r, pl.ds(Hp + 16 * j, 16)] + jnp.uint32(0x8000)
                    lo = lax.shift_right_logical(yu, jnp.uint32(16))
                    hi = yv & jnp.uint32(0xFFFF0000)
                    pack_ref[r, pl.ds(16 * j, 16)] = hi | lo
                return carry

            lax.fori_loop(0, _CHUNK, row_body, 0)

        def outer(g, carry):
            for b in range(_NBUF):
                c = g * _NBUF + b
                pltpu.make_async_copy(
                    tab_hbm.at[pl.ds(0, _CHUNK)], rows_v.at[b], gsems[b]
                ).wait()

                @pl.when(g > 0)
                def _():
                    pltpu.make_async_copy(
                        pack_v.at[b], out_hbm.at[pl.ds(0, _CHUNK)], ssems[b]
                    ).wait()

                pack_chunk(rows_v.at[b], pack_v.at[b])
                pltpu.async_copy(
                    pack_v.at[b], out_hbm.at[pl.ds(base + c * _CHUNK, _CHUNK)], ssems[b]
                )

                @pl.when(g < n_outer - 1)
                def _():
                    pltpu.async_copy(
                        tab_hbm.at[idx_v.at[pl.ds((c + _NBUF) * _CHUNK, _CHUNK)]],
                        rows_v.at[b],
                        gsems[b],
                    )

            return carry

        lax.fori_loop(0, n_outer, outer, 0)
        for b in range(_NBUF):
            pltpu.make_async_copy(
                pack_v.at[b], out_hbm.at[pl.ds(0, _CHUNK)], ssems[b]
            ).wait()

    return k(ids_flat, table)


def _tc_ln(gath, tt2, ent2, pos_emb, type_emb, entity_emb, gamma2, beta2):
    T = gath.shape[0]
    H = 2 * gath.shape[1]
    S = pos_emb.shape[0]
    n_s = S // _BT
    n_b = T // S
    inv_h = 1.0 / H

    def body(g_ref, tt_ref, ent_ref, pos_ref, te_ref, ee_ref, ga_ref, be_ref, o_ref):
        comb8 = jnp.concatenate(
            [te_ref[0, :][None, :] + ee_ref[...], te_ref[1, :][None, :] + ee_ref[...]],
            axis=0,
        )
        idx8 = tt_ref[...] * 4 + ent_ref[...]
        onehot = (idx8 == lax.broadcasted_iota(jnp.int32, (1, 8), 1)).astype(jnp.float32)
        w = g_ref[...]
        xa = lax.bitcast_convert_type(lax.shift_left(w, jnp.uint32(16)), jnp.float32)
        xb = lax.bitcast_convert_type(w & jnp.uint32(0xFFFF0000), jnp.float32)
        x = jnp.concatenate([xa, xb], axis=1) + pos_ref[...] + jnp.dot(
            onehot, comb8, preferred_element_type=jnp.float32
        )
        ones = jnp.ones((H, 1), dtype=jnp.float32)
        mean = jnp.dot(x, ones, preferred_element_type=jnp.float32) * inv_h
        m2 = jnp.dot(x * x, ones, preferred_element_type=jnp.float32) * inv_h
        var = m2 - mean * mean
        rstd = lax.rsqrt(var + EPS)
        scale = rstd * ga_ref[...]
        shift = be_ref[...] - mean * scale
        o_ref[...] = x * scale + shift

    return pl.pallas_call(
        body,
        grid=(n_s, n_b),
        in_specs=[
            pl.BlockSpec((_BT, H // 2), lambda s, b: (b * n_s + s, 0)),
            pl.BlockSpec((_BT, 1), lambda s, b: (b * n_s + s, 0)),
            pl.BlockSpec((_BT, 1), lambda s, b: (b * n_s + s, 0)),
            pl.BlockSpec((_BT, H), lambda s, b: (s, 0)),
            pl.BlockSpec((2, H), lambda s, b: (0, 0)),
            pl.BlockSpec((4, H), lambda s, b: (0, 0)),
            pl.BlockSpec((1, H), lambda s, b: (0, 0)),
            pl.BlockSpec((1, H), lambda s, b: (0, 0)),
        ],
        out_specs=pl.BlockSpec((_BT, H), lambda s, b: (b * n_s + s, 0)),
        out_shape=jax.ShapeDtypeStruct((T, H), jnp.float32),
    )(gath, tt2, ent2, pos_emb, type_emb, entity_emb, gamma2, beta2)


def kernel(input_ids, entity_ids, token_type_ids, word_emb, pos_emb, type_emb, entity_emb, gamma, beta):
    B, S = input_ids.shape
    H = word_emb.shape[1]
    T = B * S
    ids = input_ids.reshape(T).astype(jnp.int32)
    gath = _sc_gather_pack(ids, jax.lax.bitcast_convert_type(word_emb, jnp.uint32))
    tt2 = token_type_ids.reshape(T, 1).astype(jnp.int32)
    ent2 = entity_ids.reshape(T, 1).astype(jnp.int32)
    out = _tc_ln(
        gath, tt2, ent2, pos_emb, type_emb, entity_emb,
        gamma.reshape(1, H), beta.reshape(1, H),
    )
    return out.reshape(B, S, H)


# speedup vs baseline: 1.6114x; 1.6114x over previous
import functools

import jax
import jax.numpy as jnp
from jax import lax
from jax.experimental import pallas as pl
from jax.experimental.pallas import tpu as pltpu
from jax.experimental.pallas import tpu_sc as plsc

EPS = 1e-12

_NC = 2
_NS = 16
_NW = _NC * _NS
_CHUNK = 32
_NBUF = 4
_BT = 2048


def _sc_gather(ids_flat, table):
    T = ids_flat.shape[0]
    H = table.shape[1]
    per_w = T // _NW
    n_chunks = per_w // _CHUNK
    mesh = plsc.VectorSubcoreMesh(core_axis_name="c", subcore_axis_name="s")

    @functools.partial(
        pl.kernel,
        out_type=jax.ShapeDtypeStruct((T, H), jnp.float32),
        mesh=mesh,
        scratch_types=[
            pltpu.VMEM((per_w,), jnp.int32),
            pltpu.VMEM((_NBUF, _CHUNK, H), jnp.float32),
        ]
        + [pltpu.SemaphoreType.DMA] * (2 * _NBUF),
    )
    def k(ids_hbm, tab_hbm, out_hbm, idx_v, rows_v, *sems):
        wid = lax.axis_index("s") * _NC + lax.axis_index("c")
        base = wid * per_w
        gsems = sems[:_NBUF]
        ssems = sems[_NBUF:]
        gcp = [None] * _NBUF
        scp = [None] * _NBUF
        pltpu.sync_copy(ids_hbm.at[pl.ds(base, per_w)], idx_v)
        for i in range(n_chunks):
            b = i % _NBUF
            if scp[b] is not None:
                scp[b].wait()
            gcp[b] = pltpu.async_copy(
                tab_hbm.at[idx_v.at[pl.ds(i * _CHUNK, _CHUNK)]], rows_v.at[b], gsems[b]
            )
            if i >= 2:
                pb = (i - 2) % _NBUF
                gcp[pb].wait()
                scp[pb] = pltpu.async_copy(
                    rows_v.at[pb],
                    out_hbm.at[pl.ds(base + (i - 2) * _CHUNK, _CHUNK)],
                    ssems[pb],
                )
        for j in (n_chunks - 2, n_chunks - 1):
            jb = j % _NBUF
            gcp[jb].wait()
            scp[jb] = pltpu.async_copy(
                rows_v.at[jb], out_hbm.at[pl.ds(base + j * _CHUNK, _CHUNK)], ssems[jb]
            )
        for j in range(_NBUF):
            if scp[j] is not None:
                scp[j].wait()

    return k(ids_flat, table)


def _tc_ln(gath, tt2, ent2, pos_emb, type_emb, entity_emb, gamma2, beta2):
    T, H = gath.shape
    S = pos_emb.shape[0]
    n_s = S // _BT
    n_b = T // S
    inv_h = 1.0 / H

    def body(g_ref, tt_ref, ent_ref, pos_ref, te_ref, ee_ref, ga_ref, be_ref, o_ref):
        comb8 = jnp.concatenate(
            [te_ref[0, :][None, :] + ee_ref[...], te_ref[1, :][None, :] + ee_ref[...]],
            axis=0,
        )
        idx8 = tt_ref[...] * 4 + ent_ref[...]
        onehot = (idx8 == lax.broadcasted_iota(jnp.int32, (1, 8), 1)).astype(jnp.float32)
        x = g_ref[...] + pos_ref[...] + jnp.dot(
            onehot, comb8, preferred_element_type=jnp.float32
        )
        ones = jnp.ones((H, 1), dtype=jnp.float32)
        mean = jnp.dot(x, ones, preferred_element_type=jnp.float32) * inv_h
        m2 = jnp.dot(x * x, ones, preferred_element_type=jnp.float32) * inv_h
        var = m2 - mean * mean
        rstd = lax.rsqrt(var + EPS)
        scale = rstd * ga_ref[...]
        shift = be_ref[...] - mean * scale
        o_ref[...] = x * scale + shift

    return pl.pallas_call(
        body,
        grid=(n_s, n_b),
        in_specs=[
            pl.BlockSpec((_BT, H), lambda s, b: (b * n_s + s, 0)),
            pl.BlockSpec((_BT, 1), lambda s, b: (b * n_s + s, 0)),
            pl.BlockSpec((_BT, 1), lambda s, b: (b * n_s + s, 0)),
            pl.BlockSpec((_BT, H), lambda s, b: (s, 0)),
            pl.BlockSpec((2, H), lambda s, b: (0, 0)),
            pl.BlockSpec((4, H), lambda s, b: (0, 0)),
            pl.BlockSpec((1, H), lambda s, b: (0, 0)),
            pl.BlockSpec((1, H), lambda s, b: (0, 0)),
        ],
        out_specs=pl.BlockSpec((_BT, H), lambda s, b: (b * n_s + s, 0)),
        out_shape=jax.ShapeDtypeStruct((T, H), jnp.float32),
    )(gath, tt2, ent2, pos_emb, type_emb, entity_emb, gamma2, beta2)


def kernel(input_ids, entity_ids, token_type_ids, word_emb, pos_emb, type_emb, entity_emb, gamma, beta):
    B, S = input_ids.shape
    H = word_emb.shape[1]
    T = B * S
    ids = input_ids.reshape(T).astype(jnp.int32)
    gath = _sc_gather(ids, word_emb)
    tt2 = token_type_ids.reshape(T, 1).astype(jnp.int32)
    ent2 = entity_ids.reshape(T, 1).astype(jnp.int32)
    out = _tc_ln(
        gath, tt2, ent2, pos_emb, type_emb, entity_emb,
        gamma.reshape(1, H), beta.reshape(1, H),
    )
    return out.reshape(B, S, H)
